# baseline (speedup 1.0000x reference)
import functools

import jax
import jax.numpy as jnp
import numpy as np
from jax import lax
from jax.experimental import pallas as pl
from jax.experimental.pallas import tpu as pltpu
from jax.experimental.pallas import tpu_sc as plsc

N = 10000
E = 320000
F_IN = 128
H = 64
R = 16
NB = 30
C = 6

NBLK = 10
BN = N // NBLK

NC = 2
NS = 16
NW = NC * NS
EPW = E // NW
CH = 400
NCHUNK = EPW // CH
NPT = 624
NREM = N - NS * NPT

EROW = 2500
EBLK = 250

_COLPERM = np.empty(R * H, np.int32)
for _g in range(R * H // 32):
    for _k in range(16):
        _COLPERM[_g * 32 + 2 * _k] = _g * 32 + _k
        _COLPERM[_g * 32 + 2 * _k + 1] = _g * 32 + 16 + _k


def _prep_body(att_ref, basis_ref, w_ref):
    w_ref[...] = jnp.dot(att_ref[...], basis_ref[...],
                         preferred_element_type=jnp.float32)


def _build_prep(att, basis_flat):
    return pl.pallas_call(
        _prep_body,
        out_shape=jax.ShapeDtypeStruct((R, F_IN * H), jnp.float32),
    )(att, basis_flat)


def _xr_body(x_ref, w_ref, xr_ref):
    xr_ref[...] = jnp.dot(x_ref[...], w_ref[...],
                          preferred_element_type=jnp.float32
                          ).astype(jnp.bfloat16)


XBN = 2000


def _build_xr(x, w_all):
    return pl.pallas_call(
        _xr_body,
        grid=(N // XBN,),
        in_specs=[
            pl.BlockSpec((XBN, F_IN), lambda j: (j, 0)),
            pl.BlockSpec((F_IN, R * H), lambda j: (0, 0)),
        ],
        out_specs=pl.BlockSpec((XBN, R * H), lambda j: (j, 0)),
        out_shape=jax.ShapeDtypeStruct((N, R * H), jnp.bfloat16),
    )(x, w_all)


def _sc_aggregate(table, edge_index, etype, norm, *, use_rel):
    mesh = plsc.VectorSubcoreMesh(core_axis_name="c", subcore_axis_name="s")

    tdt = jnp.bfloat16 if use_rel else jnp.float32
    scratch = [
        pltpu.VMEM((EPW,), jnp.int32),
        pltpu.VMEM((EPW,), jnp.int32),
        pltpu.VMEM((CH,), jnp.float32),
        pltpu.VMEM((CH,), jnp.float32),
        pltpu.VMEM((CH, H), tdt),
        pltpu.VMEM((CH, H), tdt),
        pltpu.VMEM((CH, H), jnp.float32) if use_rel else None,
        pltpu.VMEM_SHARED((N, H), jnp.float32),
        pltpu.SemaphoreType.DMA,
        pltpu.SemaphoreType.DMA,
        pltpu.SemaphoreType.DMA,
        pltpu.SemaphoreType.DMA,
    ]
    @functools.partial(
        pl.kernel,
        out_type=jax.ShapeDtypeStruct((NC, N, H), jnp.float32),
        mesh=mesh,
        scratch_types=scratch,
        compiler_params=pltpu.CompilerParams(use_tc_tiling_on_sc=False,
                                             needs_layout_passes=False),
    )
    def k(table_h, ei_h, et_h, norm_h, out_h,
          gidx_v, dst_v, norm0, norm1, rows0, rows1, srows, acc_sh,
          sem0, sem1, semn0, semn1):
        if not use_rel:
            srows = rows0
        c = lax.axis_index("c")
        s = lax.axis_index("s")
        wid = c * NS + s
        e0 = wid * EPW

        pltpu.sync_copy(ei_h.at[0, pl.ds(e0, EPW)], gidx_v)
        if use_rel:
            pltpu.sync_copy(et_h.at[pl.ds(e0, EPW)], dst_v)

            @plsc.parallel_loop(0, EPW // 16, 1, unroll=4)
            def g16(i):
                sl = pl.ds(i * 16, 16)
                gidx_v[sl] = gidx_v[sl] * R + dst_v[sl]
        pltpu.sync_copy(ei_h.at[1, pl.ds(e0, EPW)], dst_v)

        rows = (rows0, rows1)
        sems = (sem0, sem1)
        norms = (norm0, norm1)
        semns = (semn0, semn1)

        tbl = table_h

        def fire(ci, b):
            pltpu.async_copy(tbl.at[gidx_v.at[pl.ds(ci * CH, CH)]],
                             rows[b], sems[b])
            if use_rel:
                pltpu.async_copy(norm_h.at[pl.ds(e0 + ci * CH, CH)],
                                 norms[b], semns[b])

        def drain(b):
            pltpu.make_async_copy(tbl.at[pl.ds(0, CH)], rows[b],
                                  sems[b]).wait()
            if use_rel:
                pltpu.make_async_copy(norm_h.at[pl.ds(0, CH)], norms[b],
                                      semns[b]).wait()

        base = s * NPT

        @plsc.parallel_loop(0, CH, 1, unroll=4)
        def zr(i):
            z = jnp.zeros((16,), jnp.float32)
            for j in range(H // 16):
                srows[i, pl.ds(j * 16, 16)] = z
        pltpu.sync_copy(srows.at[pl.ds(0, CH)], acc_sh.at[pl.ds(base, CH)])
        pltpu.sync_copy(srows.at[pl.ds(0, NPT - CH)],
                        acc_sh.at[pl.ds(base + CH, NPT - CH)])

        @pl.when(s == NS - 1)
        def _():
            pltpu.sync_copy(srows.at[pl.ds(0, NREM)],
                            acc_sh.at[pl.ds(NS * NPT, NREM)])

        fire(0, 0)
        fire(1, 1)
        plsc.subcore_barrier()

        def scale(buf, nbuf, ci):
            @plsc.parallel_loop(0, CH // 16, 1, unroll=2)
            def s16(i):
                ob = i * 16
                nv = nbuf[pl.ds(ob, 16)]
                for e16 in range(16):
                    cidx = jnp.full((16, 1), e16, jnp.int32)
                    ns = lax.gather(
                        nv, cidx,
                        lax.GatherDimensionNumbers(
                            offset_dims=(), collapsed_slice_dims=(0,),
                            start_index_map=(0,)),
                        (1,), mode=lax.GatherScatterMode.PROMISE_IN_BOUNDS)
                    for j in range(H // 32):
                        r2 = buf[ob + e16, pl.ds(j * 32, 32)]
                        va, vb = plsc.unpack(
                            r2, format=plsc.PackFormat.INTERLEAVED)
                        srows[ob + e16, pl.ds(j * 32, 16)] = va * ns
                        srows[ob + e16, pl.ds(j * 32 + 16, 16)] = vb * ns

        def process(ci, b, do_fire):
            drain(b)
            if use_rel:
                scale(rows[b], norms[b], ci)
            sc_src = srows if use_rel else rows[b]
            pltpu.sync_copy(sc_src, acc_sh.at[dst_v.at[pl.ds(ci * CH, CH)]],
                            add=True)
            if do_fire is not None:
                @pl.when(do_fire)
                def _():
                    fire(ci + 2, b)

        def pair(i, _):
            ci0 = i * 2
            process(ci0, 0, ci0 + 2 < NCHUNK)
            process(ci0 + 1, 1, ci0 + 3 < NCHUNK)
            return 0
        lax.fori_loop(0, NCHUNK // 2, pair, 0)
        process(NCHUNK - 1, 0, None)

        plsc.subcore_barrier()

        pltpu.sync_copy(acc_sh.at[pl.ds(base, CH)], srows)
        pltpu.sync_copy(srows, out_h.at[c, pl.ds(base, CH)])
        pltpu.sync_copy(acc_sh.at[pl.ds(base + CH, NPT - CH)],
                        srows.at[pl.ds(0, NPT - CH)])
        pltpu.sync_copy(srows.at[pl.ds(0, NPT - CH)],
                        out_h.at[c, pl.ds(base + CH, NPT - CH)])

        @pl.when(s == NS - 1)
        def _():
            pltpu.sync_copy(acc_sh.at[pl.ds(NS * NPT, NREM)],
                            srows.at[pl.ds(0, NREM)])
            pltpu.sync_copy(srows.at[pl.ds(0, NREM)],
                            out_h.at[c, pl.ds(NS * NPT, NREM)])

    return k(table, edge_index, etype, norm)


def _mid_body(agg1_ref, x_ref, root_ref, b1_ref, wn_ref, wr_ref, bg_ref,
              h_ref, rp2_ref):
    out1 = (agg1_ref[0] + agg1_ref[1]
            + jnp.dot(x_ref[...], root_ref[...],
                      preferred_element_type=jnp.float32)
            + b1_ref[...])
    h_ref[...] = jnp.dot(out1, wn_ref[...], preferred_element_type=jnp.float32)
    rp2_ref[...] = (jnp.dot(out1, wr_ref[...],
                            preferred_element_type=jnp.float32)
                    + bg_ref[...])


def _build_mid(agg1p, x, root, b1, wn, wr, bg):
    return pl.pallas_call(
        _mid_body,
        grid=(NBLK,),
        in_specs=[
            pl.BlockSpec((NC, BN, H), lambda j: (0, j, 0)),
            pl.BlockSpec((BN, F_IN), lambda j: (j, 0)),
            pl.BlockSpec((F_IN, H), lambda j: (0, 0)),
            pl.BlockSpec((1, H), lambda j: (0, 0)),
            pl.BlockSpec((H, H), lambda j: (0, 0)),
            pl.BlockSpec((H, H), lambda j: (0, 0)),
            pl.BlockSpec((1, H), lambda j: (0, 0)),
        ],
        out_specs=[
            pl.BlockSpec((BN, H), lambda j: (j, 0)),
            pl.BlockSpec((BN, H), lambda j: (j, 0)),
        ],
        out_shape=[
            jax.ShapeDtypeStruct((N, H), jnp.float32),
            jax.ShapeDtypeStruct((N, H), jnp.float32),
        ],
    )(agg1p, x, root, b1, wn, wr, bg)


def _head_body(agg2_ref, rp2_ref, x_ref, wlx_ref, wlo_ref, bl_ref,
               wf_ref, bf_ref, out_ref):
    out2 = agg2_ref[0] + agg2_ref[1] + rp2_ref[...]
    hid = (jnp.dot(x_ref[...], wlx_ref[...],
                   preferred_element_type=jnp.float32)
           + jnp.dot(out2, wlo_ref[...], preferred_element_type=jnp.float32)
           + bl_ref[...])
    hid = jnp.maximum(hid, 0.0)
    lg = jnp.dot(hid, wf_ref[...], preferred_element_type=jnp.float32) \
        + bf_ref[...]
    m = jnp.max(lg, axis=1, keepdims=True)
    ssum = jnp.sum(jnp.exp(lg - m), axis=1, keepdims=True)
    out_ref[...] = lg - m - jnp.log(ssum)


def _build_head(agg2p, rp2, x, wlx, wlo, bl, wf, bf):
    return pl.pallas_call(
        _head_body,
        grid=(NBLK,),
        in_specs=[
            pl.BlockSpec((NC, BN, H), lambda j: (0, j, 0)),
            pl.BlockSpec((BN, H), lambda j: (j, 0)),
            pl.BlockSpec((BN, F_IN), lambda j: (j, 0)),
            pl.BlockSpec((F_IN, H), lambda j: (0, 0)),
            pl.BlockSpec((H, H), lambda j: (0, 0)),
            pl.BlockSpec((1, H), lambda j: (0, 0)),
            pl.BlockSpec((H, C), lambda j: (0, 0)),
            pl.BlockSpec((1, C), lambda j: (0, 0)),
        ],
        out_specs=pl.BlockSpec((BN, C), lambda j: (j, 0)),
        out_shape=jax.ShapeDtypeStruct((N, C), jnp.float32),
    )(agg2p, rp2, x, wlx, wlo, bl, wf, bf)


def kernel(x, edge_index, edge_norm, edge_type, seq_lengths, umask,
           nodal_attn, avec, basis, att, root, b1, Wg_nei, Wg_root, bg,
           Wl, bl, Wf, bf):
    basis_flat = basis.reshape(NB, F_IN * H)
    w_flat = _build_prep(att, basis_flat)
    w_all = w_flat.reshape(R, F_IN, H).transpose(1, 0, 2).reshape(
        F_IN, R * H)[:, _COLPERM]
    xr2 = _build_xr(x, w_all)
    xr_flat = xr2.reshape(N * R, H)

    agg1p = _sc_aggregate(xr_flat, edge_index, edge_type, edge_norm,
                          use_rel=True)

    h, rp2 = _build_mid(agg1p, x, root, b1.reshape(1, H), Wg_nei,
                        Wg_root, bg.reshape(1, H))

    agg2p = _sc_aggregate(h, edge_index, edge_type, edge_norm,
                          use_rel=False)

    return _build_head(agg2p, rp2, x, Wl[:F_IN], Wl[F_IN:],
                       bl.reshape(1, H), Wf, bf.reshape(1, C))


# speedup vs baseline: 1.1983x; 1.1983x over previous
import functools

import jax
import jax.numpy as jnp
import numpy as np
from jax import lax
from jax.experimental import pallas as pl
from jax.experimental.pallas import tpu as pltpu
from jax.experimental.pallas import tpu_sc as plsc

N = 10000
E = 320000
F_IN = 128
H = 64
R = 16
NB = 30
C = 6

NBLK = 10
BN = N // NBLK

NC = 2
NS = 16
NW = NC * NS
EPW = E // NW
CH = 400
NCHUNK = EPW // CH
NPT = 624
NREM = N - NS * NPT

EROW = 2500
EBLK = 250

_COLPERM = np.empty(R * H, np.int32)
for _g in range(R * H // 32):
    for _k in range(16):
        _COLPERM[_g * 32 + 2 * _k] = _g * 32 + _k
        _COLPERM[_g * 32 + 2 * _k + 1] = _g * 32 + 16 + _k


def _prep_body(att_ref, basis_ref, w_ref):
    w_ref[...] = jnp.dot(att_ref[...], basis_ref[...],
                         preferred_element_type=jnp.float32)


def _build_prep(att, basis_flat):
    return pl.pallas_call(
        _prep_body,
        out_shape=jax.ShapeDtypeStruct((R, F_IN * H), jnp.float32),
    )(att, basis_flat)


def _xr_body(x_ref, w_ref, xr_ref):
    xr_ref[...] = jnp.dot(x_ref[...], w_ref[...],
                          preferred_element_type=jnp.float32)


XBN = 2000


def _build_xr(x, w_all):
    return pl.pallas_call(
        _xr_body,
        grid=(N // XBN,),
        in_specs=[
            pl.BlockSpec((XBN, F_IN), lambda j: (j, 0)),
            pl.BlockSpec((F_IN, R * H), lambda j: (0, 0)),
        ],
        out_specs=pl.BlockSpec((XBN, R * H), lambda j: (j, 0)),
        out_shape=jax.ShapeDtypeStruct((N, R * H), jnp.float32),
    )(x, w_all)


def _sc_aggregate(table, edge_index, etype, norm, *, use_rel):
    mesh = plsc.VectorSubcoreMesh(core_axis_name="c", subcore_axis_name="s")

    tdt = jnp.float32
    scratch = [
        pltpu.VMEM((EPW,), jnp.int32),
        pltpu.VMEM((EPW,), jnp.int32),
        pltpu.VMEM((CH,), jnp.float32),
        pltpu.VMEM((CH,), jnp.float32),
        pltpu.VMEM((CH, H), tdt),
        pltpu.VMEM((CH, H), tdt),
        pltpu.VMEM_SHARED((N, H), jnp.float32),
        pltpu.SemaphoreType.DMA,
        pltpu.SemaphoreType.DMA,
        pltpu.SemaphoreType.DMA,
        pltpu.SemaphoreType.DMA,
    ]
    @functools.partial(
        pl.kernel,
        out_type=jax.ShapeDtypeStruct((NC, N, H), jnp.float32),
        mesh=mesh,
        scratch_types=scratch,
        compiler_params=pltpu.CompilerParams(use_tc_tiling_on_sc=False,
                                             needs_layout_passes=False),
    )
    def k(table_h, ei_h, et_h, norm_h, out_h,
          gidx_v, dst_v, norm0, norm1, rows0, rows1, acc_sh,
          sem0, sem1, semn0, semn1):
        srows = rows0
        c = lax.axis_index("c")
        s = lax.axis_index("s")
        wid = c * NS + s
        e0 = wid * EPW

        pltpu.sync_copy(ei_h.at[0, pl.ds(e0, EPW)], gidx_v)
        if use_rel:
            pltpu.sync_copy(et_h.at[pl.ds(e0, EPW)], dst_v)

            @plsc.parallel_loop(0, EPW // 16, 1, unroll=4)
            def g16(i):
                sl = pl.ds(i * 16, 16)
                gidx_v[sl] = gidx_v[sl] * R + dst_v[sl]
        pltpu.sync_copy(ei_h.at[1, pl.ds(e0, EPW)], dst_v)

        rows = (rows0, rows1)
        sems = (sem0, sem1)
        norms = (norm0, norm1)
        semns = (semn0, semn1)

        tbl = table_h

        def fire(ci, b):
            pltpu.async_copy(tbl.at[gidx_v.at[pl.ds(ci * CH, CH)]],
                             rows[b], sems[b])
            if use_rel:
                pltpu.async_copy(norm_h.at[pl.ds(e0 + ci * CH, CH)],
                                 norms[b], semns[b])

        def drain(b):
            pltpu.make_async_copy(tbl.at[pl.ds(0, CH)], rows[b],
                                  sems[b]).wait()
            if use_rel:
                pltpu.make_async_copy(norm_h.at[pl.ds(0, CH)], norms[b],
                                      semns[b]).wait()

        base = s * NPT

        @plsc.parallel_loop(0, CH, 1, unroll=4)
        def zr(i):
            z = jnp.zeros((16,), jnp.float32)
            for j in range(H // 16):
                srows[i, pl.ds(j * 16, 16)] = z
        pltpu.sync_copy(srows.at[pl.ds(0, CH)], acc_sh.at[pl.ds(base, CH)])
        pltpu.sync_copy(srows.at[pl.ds(0, NPT - CH)],
                        acc_sh.at[pl.ds(base + CH, NPT - CH)])

        @pl.when(s == NS - 1)
        def _():
            pltpu.sync_copy(srows.at[pl.ds(0, NREM)],
                            acc_sh.at[pl.ds(NS * NPT, NREM)])

        fire(0, 0)
        fire(1, 1)
        plsc.subcore_barrier()

        def scale(buf, nbuf, ci):
            @plsc.parallel_loop(0, CH // 16, 1, unroll=2)
            def s16(i):
                ob = i * 16
                nv = nbuf[pl.ds(ob, 16)]
                for e16 in range(16):
                    cidx = jnp.full((16, 1), e16, jnp.int32)
                    ns = lax.gather(
                        nv, cidx,
                        lax.GatherDimensionNumbers(
                            offset_dims=(), collapsed_slice_dims=(0,),
                            start_index_map=(0,)),
                        (1,), mode=lax.GatherScatterMode.PROMISE_IN_BOUNDS)
                    for j in range(H // 16):
                        sl = pl.ds(j * 16, 16)
                        buf[ob + e16, sl] = buf[ob + e16, sl] * ns

        def process(ci, b, do_fire):
            drain(b)
            if use_rel:
                scale(rows[b], norms[b], ci)
            pltpu.sync_copy(rows[b], acc_sh.at[dst_v.at[pl.ds(ci * CH, CH)]],
                            add=True)
            if do_fire is not None:
                @pl.when(do_fire)
                def _():
                    fire(ci + 2, b)

        def pair(i, _):
            ci0 = i * 2
            process(ci0, 0, ci0 + 2 < NCHUNK)
            process(ci0 + 1, 1, ci0 + 3 < NCHUNK)
            return 0
        lax.fori_loop(0, NCHUNK // 2, pair, 0)
        process(NCHUNK - 1, 0, None)

        plsc.subcore_barrier()

        pltpu.sync_copy(acc_sh.at[pl.ds(base, CH)], srows)
        pltpu.sync_copy(srows, out_h.at[c, pl.ds(base, CH)])
        pltpu.sync_copy(acc_sh.at[pl.ds(base + CH, NPT - CH)],
                        srows.at[pl.ds(0, NPT - CH)])
        pltpu.sync_copy(srows.at[pl.ds(0, NPT - CH)],
                        out_h.at[c, pl.ds(base + CH, NPT - CH)])

        @pl.when(s == NS - 1)
        def _():
            pltpu.sync_copy(acc_sh.at[pl.ds(NS * NPT, NREM)],
                            srows.at[pl.ds(0, NREM)])
            pltpu.sync_copy(srows.at[pl.ds(0, NREM)],
                            out_h.at[c, pl.ds(NS * NPT, NREM)])

    return k(table, edge_index, etype, norm)


def _mid_body(agg1_ref, x_ref, root_ref, b1_ref, wn_ref, wr_ref, bg_ref,
              h_ref, rp2_ref):
    out1 = (agg1_ref[0] + agg1_ref[1]
            + jnp.dot(x_ref[...], root_ref[...],
                      preferred_element_type=jnp.float32)
            + b1_ref[...])
    h_ref[...] = jnp.dot(out1, wn_ref[...], preferred_element_type=jnp.float32)
    rp2_ref[...] = (jnp.dot(out1, wr_ref[...],
                            preferred_element_type=jnp.float32)
                    + bg_ref[...])


def _build_mid(agg1p, x, root, b1, wn, wr, bg):
    return pl.pallas_call(
        _mid_body,
        grid=(NBLK,),
        in_specs=[
            pl.BlockSpec((NC, BN, H), lambda j: (0, j, 0)),
            pl.BlockSpec((BN, F_IN), lambda j: (j, 0)),
            pl.BlockSpec((F_IN, H), lambda j: (0, 0)),
            pl.BlockSpec((1, H), lambda j: (0, 0)),
            pl.BlockSpec((H, H), lambda j: (0, 0)),
            pl.BlockSpec((H, H), lambda j: (0, 0)),
            pl.BlockSpec((1, H), lambda j: (0, 0)),
        ],
        out_specs=[
            pl.BlockSpec((BN, H), lambda j: (j, 0)),
            pl.BlockSpec((BN, H), lambda j: (j, 0)),
        ],
        out_shape=[
            jax.ShapeDtypeStruct((N, H), jnp.float32),
            jax.ShapeDtypeStruct((N, H), jnp.float32),
        ],
    )(agg1p, x, root, b1, wn, wr, bg)


def _head_body(agg2_ref, rp2_ref, x_ref, wlx_ref, wlo_ref, bl_ref,
               wf_ref, bf_ref, out_ref):
    out2 = agg2_ref[0] + agg2_ref[1] + rp2_ref[...]
    hid = (jnp.dot(x_ref[...], wlx_ref[...],
                   preferred_element_type=jnp.float32)
           + jnp.dot(out2, wlo_ref[...], preferred_element_type=jnp.float32)
           + bl_ref[...])
    hid = jnp.maximum(hid, 0.0)
    lg = jnp.dot(hid, wf_ref[...], preferred_element_type=jnp.float32) \
        + bf_ref[...]
    m = jnp.max(lg, axis=1, keepdims=True)
    ssum = jnp.sum(jnp.exp(lg - m), axis=1, keepdims=True)
    out_ref[...] = lg - m - jnp.log(ssum)


def _build_head(agg2p, rp2, x, wlx, wlo, bl, wf, bf):
    return pl.pallas_call(
        _head_body,
        grid=(NBLK,),
        in_specs=[
            pl.BlockSpec((NC, BN, H), lambda j: (0, j, 0)),
            pl.BlockSpec((BN, H), lambda j: (j, 0)),
            pl.BlockSpec((BN, F_IN), lambda j: (j, 0)),
            pl.BlockSpec((F_IN, H), lambda j: (0, 0)),
            pl.BlockSpec((H, H), lambda j: (0, 0)),
            pl.BlockSpec((1, H), lambda j: (0, 0)),
            pl.BlockSpec((H, C), lambda j: (0, 0)),
            pl.BlockSpec((1, C), lambda j: (0, 0)),
        ],
        out_specs=pl.BlockSpec((BN, C), lambda j: (j, 0)),
        out_shape=jax.ShapeDtypeStruct((N, C), jnp.float32),
    )(agg2p, rp2, x, wlx, wlo, bl, wf, bf)


def kernel(x, edge_index, edge_norm, edge_type, seq_lengths, umask,
           nodal_attn, avec, basis, att, root, b1, Wg_nei, Wg_root, bg,
           Wl, bl, Wf, bf):
    basis_flat = basis.reshape(NB, F_IN * H)
    w_flat = _build_prep(att, basis_flat)
    w_all = w_flat.reshape(R, F_IN, H).transpose(1, 0, 2).reshape(
        F_IN, R * H)
    xr2 = _build_xr(x, w_all)
    xr_flat = xr2.reshape(N * R, H)

    agg1p = _sc_aggregate(xr_flat, edge_index, edge_type, edge_norm,
                          use_rel=True)

    h, rp2 = _build_mid(agg1p, x, root, b1.reshape(1, H), Wg_nei,
                        Wg_root, bg.reshape(1, H))

    agg2p = _sc_aggregate(h, edge_index, edge_type, edge_norm,
                          use_rel=False)

    return _build_head(agg2p, rp2, x, Wl[:F_IN], Wl[F_IN:],
                       bl.reshape(1, H), Wf, bf.reshape(1, C))
